# Initial kernel scaffold; baseline (speedup 1.0000x reference)
import jax
import jax.numpy as jnp
from jax.experimental import pallas as pl


def kernel(x, edge_index, edge_attr, We1, be1, We2, be2, Wl1, bl1, Wr1, Wl2, bl2, Wr2, Wo1, bo1, Wo2, bo2):
    raise NotImplementedError("write your pallas kernel here")


# speedup vs baseline: 3.6037x; 3.6037x over previous
import functools

import jax
import jax.numpy as jnp
from jax import lax
from jax.experimental import pallas as pl
from jax.experimental.pallas import tpu as pltpu
from jax.experimental.pallas import tpu_sc as plsc

N = 10000
E = 320000
D = 128
H = 64

NPAD = 10240
TW = 128
NTILES = 32
EPT = 10240
EPAD = EPT * NTILES
CHUNK = 128
NCHUNKS = EPT // CHUNK
IGRP = 16
NGRP = NCHUNKS // IGRP
RPT = NPAD // 16

f32 = jnp.float32


def _mktab(y):
    m = y.shape[0]
    return jnp.concatenate(
        [y, jnp.ones((m, 1), f32), jnp.zeros((m, TW - H - 1), f32)], axis=1)


def _node1_body(x_ref, wl_ref, wr_ref, bl_ref, yt_ref, r_ref):
    x = x_ref[...]
    y = jnp.dot(x, wl_ref[...], preferred_element_type=f32)
    yt_ref[...] = _mktab(y)
    r_ref[...] = jnp.dot(x, wr_ref[...], preferred_element_type=f32) + bl_ref[...]


def _node1(xp, Wl1, Wr1, bl1):
    G = NPAD // 256
    return pl.pallas_call(
        _node1_body,
        grid=(G,),
        in_specs=[
            pl.BlockSpec((256, D), lambda i: (i, 0)),
            pl.BlockSpec((D, H), lambda i: (0, 0)),
            pl.BlockSpec((D, H), lambda i: (0, 0)),
            pl.BlockSpec((1, H), lambda i: (0, 0)),
        ],
        out_specs=[
            pl.BlockSpec((256, TW), lambda i: (i, 0)),
            pl.BlockSpec((256, H), lambda i: (i, 0)),
        ],
        out_shape=[
            jax.ShapeDtypeStruct((NPAD, TW), f32),
            jax.ShapeDtypeStruct((NPAD, H), f32),
        ],
    )(xp, Wl1, Wr1, bl1)


def _comb_body(a0_ref, a1_ref, r_ref, wl_ref, wr_ref, bl_ref, yt_ref, r2_ref):
    a = a0_ref[...] + a1_ref[...]
    deg = jnp.maximum(a[:, H:H + 1], 1.0)
    h = jnp.maximum(a[:, :H] / deg + r_ref[...], 0.0)
    y2 = jnp.dot(h, wl_ref[...], preferred_element_type=f32)
    yt_ref[...] = _mktab(y2)
    r2_ref[...] = jnp.dot(h, wr_ref[...], preferred_element_type=f32) + bl_ref[...]


def _comb1(acc, r1, Wl2, Wr2, bl2):
    G = NPAD // 256
    return pl.pallas_call(
        _comb_body,
        grid=(G,),
        in_specs=[
            pl.BlockSpec((256, TW), lambda i: (i, 0)),
            pl.BlockSpec((256, TW), lambda i: (i + NPAD // 256, 0)),
            pl.BlockSpec((256, H), lambda i: (i, 0)),
            pl.BlockSpec((H, H), lambda i: (0, 0)),
            pl.BlockSpec((H, H), lambda i: (0, 0)),
            pl.BlockSpec((1, H), lambda i: (0, 0)),
        ],
        out_specs=[
            pl.BlockSpec((256, TW), lambda i: (i, 0)),
            pl.BlockSpec((256, H), lambda i: (i, 0)),
        ],
        out_shape=[
            jax.ShapeDtypeStruct((NPAD, TW), f32),
            jax.ShapeDtypeStruct((NPAD, H), f32),
        ],
    )(acc, acc, r1, Wl2, Wr2, bl2)


def _comb2_body(a0_ref, a1_ref, r_ref, wo_ref, g_ref):
    a = a0_ref[...] + a1_ref[...]
    deg = jnp.maximum(a[:, H:H + 1], 1.0)
    h = jnp.maximum(a[:, :H] / deg + r_ref[...], 0.0)
    g = jnp.dot(h, wo_ref[...], preferred_element_type=f32)
    g_ref[...] = jnp.concatenate([g, jnp.zeros((g.shape[0], TW - H), f32)], axis=1)


def _comb2(acc, r2, Wo1):
    G = NPAD // 256
    return pl.pallas_call(
        _comb2_body,
        grid=(G,),
        in_specs=[
            pl.BlockSpec((256, TW), lambda i: (i, 0)),
            pl.BlockSpec((256, TW), lambda i: (i + NPAD // 256, 0)),
            pl.BlockSpec((256, H), lambda i: (i, 0)),
            pl.BlockSpec((H, H), lambda i: (0, 0)),
        ],
        out_specs=pl.BlockSpec((256, TW), lambda i: (i, 0)),
        out_shape=jax.ShapeDtypeStruct((NPAD, TW), f32),
    )(acc, acc, r2, Wo1)


def _final_body(ea_ref, efp_ref, we1_ref, be1_ref, we2_ref, wo1_ref,
                bo1_ref, be2_ref, wo2_ref, bo2_ref, o_ref):
    t = jnp.maximum(
        jnp.dot(ea_ref[...], we1_ref[...], preferred_element_type=f32)
        + be1_ref[...], 0.0)
    w2o = jnp.dot(we2_ref[...], wo1_ref[...], preferred_element_type=f32)
    cvec = jnp.dot(be2_ref[...], wo1_ref[...], preferred_element_type=f32) + bo1_ref[...]
    z = jnp.maximum(
        jnp.dot(t, w2o, preferred_element_type=f32)
        + efp_ref[...][:, :H] + cvec, 0.0)
    o = jnp.dot(z, wo2_ref[...], preferred_element_type=f32) + bo2_ref[...]
    o_ref[...] = jax.nn.sigmoid(o)


def _final(edge_attr, efp, We1, be1, We2, Wo1, bo1, be2, Wo2, bo2):
    BE = 512
    G = E // BE
    return pl.pallas_call(
        _final_body,
        grid=(G,),
        in_specs=[
            pl.BlockSpec((BE, 16), lambda i: (i, 0)),
            pl.BlockSpec((BE, TW), lambda i: (i, 0)),
            pl.BlockSpec((16, H), lambda i: (0, 0)),
            pl.BlockSpec((1, H), lambda i: (0, 0)),
            pl.BlockSpec((H, H), lambda i: (0, 0)),
            pl.BlockSpec((H, H), lambda i: (0, 0)),
            pl.BlockSpec((1, H), lambda i: (0, 0)),
            pl.BlockSpec((1, H), lambda i: (0, 0)),
            pl.BlockSpec((H, 1), lambda i: (0, 0)),
            pl.BlockSpec((1, 1), lambda i: (0, 0)),
        ],
        out_specs=pl.BlockSpec((BE, 1), lambda i: (i, 0)),
        out_shape=jax.ShapeDtypeStruct((E, 1), f32),
    )(edge_attr, efp, We1, be1, We2, Wo1, bo1, be2, Wo2, bo2)


_MESH = dict(core_axis_name="c", subcore_axis_name="s")


def _seg_sum(table, srcm, dstm, ztab):
    mesh = plsc.VectorSubcoreMesh(**_MESH)

    @functools.partial(
        pl.kernel,
        out_type=jax.ShapeDtypeStruct((2 * NPAD, TW), f32),
        mesh=mesh,
        scratch_types=[
            pltpu.VMEM_SHARED((NPAD, TW), f32),
            pltpu.VMEM((IGRP, CHUNK), jnp.int32),
            pltpu.VMEM((IGRP, CHUNK), jnp.int32),
            pltpu.VMEM((CHUNK, TW), f32),
            pltpu.SemaphoreType.DMA,
        ],
    )
    def k(table_hbm, src_hbm, dst_hbm, z_hbm, out_hbm,
          acc_sh, sidx, didx, rows, sem):
        c = lax.axis_index("c")
        s = lax.axis_index("s")
        r0 = s * RPT
        pltpu.sync_copy(z_hbm.at[pl.ds(r0, RPT)], acc_sh.at[pl.ds(r0, RPT)])
        wid = c * 16 + s
        cb = wid * NCHUNKS
        plsc.subcore_barrier()

        def grp(gi, carry):
            pltpu.sync_copy(src_hbm.at[pl.ds(cb + gi * IGRP, IGRP)], sidx)
            pltpu.sync_copy(dst_hbm.at[pl.ds(cb + gi * IGRP, IGRP)], didx)

            def body(j, carry2):
                pltpu.async_copy(table_hbm.at[sidx.at[j]], rows, sem).wait()
                pltpu.sync_copy(rows, acc_sh.at[didx.at[j]], add=True)
                return carry2

            lax.fori_loop(0, IGRP, body, 0)
            return carry

        lax.fori_loop(0, NGRP, grp, 0)
        plsc.subcore_barrier()
        pltpu.sync_copy(acc_sh.at[pl.ds(r0, RPT)],
                        out_hbm.at[pl.ds(c * NPAD + r0, RPT)])

    return k(table, srcm, dstm, ztab)


def _edge_feats(g, srcm, dstm):
    mesh = plsc.VectorSubcoreMesh(**_MESH)

    @functools.partial(
        pl.kernel,
        out_type=jax.ShapeDtypeStruct((EPAD, TW), f32),
        mesh=mesh,
        scratch_types=[
            pltpu.VMEM((IGRP, CHUNK), jnp.int32),
            pltpu.VMEM((IGRP, CHUNK), jnp.int32),
            pltpu.VMEM((CHUNK, TW), f32),
            pltpu.VMEM((CHUNK, TW), f32),
            pltpu.SemaphoreType.DMA,
            pltpu.SemaphoreType.DMA,
        ],
    )
    def k(g_hbm, src_hbm, dst_hbm, out_hbm,
          sidx, didx, ra, rb, sem_a, sem_b):
        c = lax.axis_index("c")
        s = lax.axis_index("s")
        wid = c * 16 + s
        cb = wid * NCHUNKS
        e0 = wid * EPT

        def grp(gi, carry):
            pltpu.sync_copy(src_hbm.at[pl.ds(cb + gi * IGRP, IGRP)], sidx)
            pltpu.sync_copy(dst_hbm.at[pl.ds(cb + gi * IGRP, IGRP)], didx)

            def body(j, carry2):
                ca = pltpu.async_copy(g_hbm.at[sidx.at[j]], ra, sem_a)
                cb2 = pltpu.async_copy(g_hbm.at[didx.at[j]], rb, sem_b)
                ca.wait()
                cb2.wait()

                def row(i, carry3):
                    for q in range(H // 16):
                        av = ra[i, pl.ds(q * 16, 16)]
                        bv = rb[i, pl.ds(q * 16, 16)]
                        ra[i, pl.ds(q * 16, 16)] = av + bv
                    return carry3

                lax.fori_loop(0, CHUNK, row, 0)
                pltpu.sync_copy(
                    ra, out_hbm.at[pl.ds(e0 + (gi * IGRP + j) * CHUNK, CHUNK)])
                return carry2

            lax.fori_loop(0, IGRP, body, 0)
            return carry

        lax.fori_loop(0, NGRP, grp, 0)

    return k(g, srcm, dstm)


def kernel(x, edge_index, edge_attr, We1, be1, We2, be2,
           Wl1, bl1, Wr1, Wl2, bl2, Wr2, Wo1, bo1, Wo2, bo2):
    src = edge_index[0].astype(jnp.int32)
    dst = edge_index[1].astype(jnp.int32)
    npadv = EPAD - E
    padv = N + (jnp.arange(npadv, dtype=jnp.int32) % 16)
    srcm = jnp.concatenate([src, padv]).reshape(EPAD // CHUNK, CHUNK)
    dstm = jnp.concatenate([dst, padv]).reshape(EPAD // CHUNK, CHUNK)

    xp = jnp.pad(x, ((0, NPAD - N), (0, 0)))
    ztab = jnp.zeros((NPAD, TW), f32)

    bl1r = bl1.reshape(1, H)
    bl2r = bl2.reshape(1, H)
    be1r = be1.reshape(1, H)
    be2r = be2.reshape(1, H)
    bo1r = bo1.reshape(1, H)
    bo2r = bo2.reshape(1, 1)

    y1t, r1 = _node1(xp, Wl1, Wr1, bl1r)
    acc1 = _seg_sum(y1t, srcm, dstm, ztab)
    y2t, r2 = _comb1(acc1, r1, Wl2, Wr2, bl2r)
    acc2 = _seg_sum(y2t, srcm, dstm, ztab)
    g = _comb2(acc2, r2, Wo1)
    efp = _edge_feats(g, srcm, dstm)
    out = _final(edge_attr, efp, We1, be1r, We2, Wo1, bo1r, be2r, Wo2, bo2r)
    return out[:, 0]


# speedup vs baseline: 4.8897x; 1.3569x over previous
import functools

import jax
import jax.numpy as jnp
from jax import lax
from jax.experimental import pallas as pl
from jax.experimental.pallas import tpu as pltpu
from jax.experimental.pallas import tpu_sc as plsc

N = 10000
E = 320000
D = 128
H = 64

NPAD = 10240
TW = 128
NTILES = 32
EPT = 10240
EPAD = EPT * NTILES
RPT = NPAD // 16

SCH = 64
SNC = EPT // SCH
SIG = 16
SNG = SNC // SIG

ECH = 128
ENC = EPT // ECH
EIG = 16
ENG = ENC // EIG

f32 = jnp.float32


def _mktab(y):
    m = y.shape[0]
    return jnp.concatenate(
        [y, jnp.ones((m, 1), f32), jnp.zeros((m, TW - H - 1), f32)], axis=1)


def _node1_body(x_ref, wl_ref, wr_ref, bl_ref, yt_ref, r_ref):
    x = x_ref[...]
    y = jnp.dot(x, wl_ref[...], preferred_element_type=f32)
    yt_ref[...] = _mktab(y)
    r_ref[...] = jnp.dot(x, wr_ref[...], preferred_element_type=f32) + bl_ref[...]


def _node1(xp, Wl1, Wr1, bl1):
    G = NPAD // 512
    return pl.pallas_call(
        _node1_body,
        grid=(G,),
        in_specs=[
            pl.BlockSpec((512, D), lambda i: (i, 0)),
            pl.BlockSpec((D, H), lambda i: (0, 0)),
            pl.BlockSpec((D, H), lambda i: (0, 0)),
            pl.BlockSpec((1, H), lambda i: (0, 0)),
        ],
        out_specs=[
            pl.BlockSpec((512, TW), lambda i: (i, 0)),
            pl.BlockSpec((512, H), lambda i: (i, 0)),
        ],
        out_shape=[
            jax.ShapeDtypeStruct((NPAD, TW), f32),
            jax.ShapeDtypeStruct((NPAD, H), f32),
        ],
    )(xp, Wl1, Wr1, bl1)


def _comb_body(a0_ref, a1_ref, r_ref, wl_ref, wr_ref, bl_ref, yt_ref, r2_ref):
    a = a0_ref[...] + a1_ref[...]
    deg = jnp.maximum(a[:, H:H + 1], 1.0)
    h = jnp.maximum(a[:, :H] / deg + r_ref[...], 0.0)
    y2 = jnp.dot(h, wl_ref[...], preferred_element_type=f32)
    yt_ref[...] = _mktab(y2)
    r2_ref[...] = jnp.dot(h, wr_ref[...], preferred_element_type=f32) + bl_ref[...]


def _comb1(acc, r1, Wl2, Wr2, bl2):
    G = NPAD // 512
    return pl.pallas_call(
        _comb_body,
        grid=(G,),
        in_specs=[
            pl.BlockSpec((512, TW), lambda i: (i, 0)),
            pl.BlockSpec((512, TW), lambda i: (i + NPAD // 512, 0)),
            pl.BlockSpec((512, H), lambda i: (i, 0)),
            pl.BlockSpec((H, H), lambda i: (0, 0)),
            pl.BlockSpec((H, H), lambda i: (0, 0)),
            pl.BlockSpec((1, H), lambda i: (0, 0)),
        ],
        out_specs=[
            pl.BlockSpec((512, TW), lambda i: (i, 0)),
            pl.BlockSpec((512, H), lambda i: (i, 0)),
        ],
        out_shape=[
            jax.ShapeDtypeStruct((NPAD, TW), f32),
            jax.ShapeDtypeStruct((NPAD, H), f32),
        ],
    )(acc, acc, r1, Wl2, Wr2, bl2)


def _comb2_body(a0_ref, a1_ref, r_ref, wo_ref, g_ref):
    a = a0_ref[...] + a1_ref[...]
    deg = jnp.maximum(a[:, H:H + 1], 1.0)
    h = jnp.maximum(a[:, :H] / deg + r_ref[...], 0.0)
    g = jnp.dot(h, wo_ref[...], preferred_element_type=f32)
    g_ref[...] = jnp.concatenate([g, jnp.zeros((g.shape[0], TW - H), f32)], axis=1)


def _comb2(acc, r2, Wo1):
    G = NPAD // 512
    return pl.pallas_call(
        _comb2_body,
        grid=(G,),
        in_specs=[
            pl.BlockSpec((512, TW), lambda i: (i, 0)),
            pl.BlockSpec((512, TW), lambda i: (i + NPAD // 512, 0)),
            pl.BlockSpec((512, H), lambda i: (i, 0)),
            pl.BlockSpec((H, H), lambda i: (0, 0)),
        ],
        out_specs=pl.BlockSpec((512, TW), lambda i: (i, 0)),
        out_shape=jax.ShapeDtypeStruct((NPAD, TW), f32),
    )(acc, acc, r2, Wo1)


def _final_body(ea_ref, efp_ref, we1_ref, be1_ref, we2_ref, wo1_ref,
                bo1_ref, be2_ref, wo2_ref, bo2_ref, o_ref):
    t = jnp.maximum(
        jnp.dot(ea_ref[...], we1_ref[...], preferred_element_type=f32)
        + be1_ref[...], 0.0)
    w2o = jnp.dot(we2_ref[...], wo1_ref[...], preferred_element_type=f32)
    cvec = jnp.dot(be2_ref[...], wo1_ref[...], preferred_element_type=f32) + bo1_ref[...]
    z = jnp.maximum(
        jnp.dot(t, w2o, preferred_element_type=f32) + efp_ref[...] + cvec, 0.0)
    o = jnp.dot(z, wo2_ref[...], preferred_element_type=f32) + bo2_ref[...]
    o_ref[...] = jax.nn.sigmoid(o)


def _final(edge_attr, efp, We1, be1, We2, Wo1, bo1, be2, Wo2, bo2):
    BE = 2560
    G = E // BE
    return pl.pallas_call(
        _final_body,
        grid=(G,),
        in_specs=[
            pl.BlockSpec((BE, 16), lambda i: (i, 0)),
            pl.BlockSpec((BE, H), lambda i: (i, 0)),
            pl.BlockSpec((16, H), lambda i: (0, 0)),
            pl.BlockSpec((1, H), lambda i: (0, 0)),
            pl.BlockSpec((H, H), lambda i: (0, 0)),
            pl.BlockSpec((H, H), lambda i: (0, 0)),
            pl.BlockSpec((1, H), lambda i: (0, 0)),
            pl.BlockSpec((1, H), lambda i: (0, 0)),
            pl.BlockSpec((H, 1), lambda i: (0, 0)),
            pl.BlockSpec((1, 1), lambda i: (0, 0)),
        ],
        out_specs=pl.BlockSpec((BE, 1), lambda i: (i, 0)),
        out_shape=jax.ShapeDtypeStruct((E, 1), f32),
    )(edge_attr, efp, We1, be1, We2, Wo1, bo1, be2, Wo2, bo2)


_MESH = dict(core_axis_name="c", subcore_axis_name="s")


def _seg_sum(table, srcm, dstm, ztab):
    mesh = plsc.VectorSubcoreMesh(**_MESH)

    @functools.partial(
        pl.kernel,
        out_type=jax.ShapeDtypeStruct((2 * NPAD, TW), f32),
        mesh=mesh,
        scratch_types=[
            pltpu.VMEM_SHARED((NPAD, TW), f32),
            pltpu.VMEM((SIG, SCH), jnp.int32),
            pltpu.VMEM((SIG, SCH), jnp.int32),
            pltpu.VMEM((SCH, TW), f32),
            pltpu.VMEM((SCH, TW), f32),
            pltpu.SemaphoreType.DMA,
            pltpu.SemaphoreType.DMA,
        ],
    )
    def k(table_hbm, src_hbm, dst_hbm, z_hbm, out_hbm,
          acc_sh, sidx, didx, rows0, rows1, sem0, sem1):
        c = lax.axis_index("c")
        s = lax.axis_index("s")
        r0 = s * RPT
        pltpu.sync_copy(z_hbm.at[pl.ds(r0, RPT)], acc_sh.at[pl.ds(r0, RPT)])
        wid = c * 16 + s
        cb = wid * SNC
        plsc.subcore_barrier()

        rbufs = (rows0, rows1)
        sems = (sem0, sem1)

        def grp(gi, carry):
            pltpu.sync_copy(src_hbm.at[pl.ds(cb + gi * SIG, SIG)], sidx)
            pltpu.sync_copy(dst_hbm.at[pl.ds(cb + gi * SIG, SIG)], didx)
            copies = [None, None]
            copies[0] = pltpu.async_copy(
                table_hbm.at[sidx.at[0]], rbufs[0], sems[0])
            for j in range(SIG):
                p = j % 2
                copies[p].wait()
                if j + 1 < SIG:
                    q = (j + 1) % 2
                    copies[q] = pltpu.async_copy(
                        table_hbm.at[sidx.at[j + 1]], rbufs[q], sems[q])
                pltpu.sync_copy(rbufs[p], acc_sh.at[didx.at[j]], add=True)
            return carry

        lax.fori_loop(0, SNG, grp, 0)
        plsc.subcore_barrier()
        pltpu.sync_copy(acc_sh.at[pl.ds(r0, RPT)],
                        out_hbm.at[pl.ds(c * NPAD + r0, RPT)])

    return k(table, srcm, dstm, ztab)


def _edge_feats(g, srcm, dstm):
    mesh = plsc.VectorSubcoreMesh(**_MESH)

    @functools.partial(
        pl.kernel,
        out_type=jax.ShapeDtypeStruct((EPAD, H), f32),
        mesh=mesh,
        scratch_types=[
            pltpu.VMEM((EIG, ECH), jnp.int32),
            pltpu.VMEM((EIG, ECH), jnp.int32),
            pltpu.VMEM((ECH, TW), f32),
            pltpu.VMEM((ECH, TW), f32),
            pltpu.VMEM((ECH, TW), f32),
            pltpu.VMEM((ECH, TW), f32),
            pltpu.VMEM((ECH, H), f32),
            pltpu.SemaphoreType.DMA,
            pltpu.SemaphoreType.DMA,
            pltpu.SemaphoreType.DMA,
            pltpu.SemaphoreType.DMA,
        ],
    )
    def k(g_hbm, src_hbm, dst_hbm, out_hbm,
          sidx, didx, ra0, rb0, ra1, rb1, rc, sa0, sb0, sa1, sb1):
        c = lax.axis_index("c")
        s = lax.axis_index("s")
        wid = c * 16 + s
        cb = wid * ENC
        e0 = wid * EPT

        ras = (ra0, ra1)
        rbs = (rb0, rb1)
        sas = (sa0, sa1)
        sbs = (sb0, sb1)

        def grp(gi, carry):
            pltpu.sync_copy(src_hbm.at[pl.ds(cb + gi * EIG, EIG)], sidx)
            pltpu.sync_copy(dst_hbm.at[pl.ds(cb + gi * EIG, EIG)], didx)
            ca = [None, None]
            cbq = [None, None]
            ca[0] = pltpu.async_copy(g_hbm.at[sidx.at[0]], ras[0], sas[0])
            cbq[0] = pltpu.async_copy(g_hbm.at[didx.at[0]], rbs[0], sbs[0])
            for j in range(EIG):
                p = j % 2
                ca[p].wait()
                cbq[p].wait()
                if j + 1 < EIG:
                    q = (j + 1) % 2
                    ca[q] = pltpu.async_copy(
                        g_hbm.at[sidx.at[j + 1]], ras[q], sas[q])
                    cbq[q] = pltpu.async_copy(
                        g_hbm.at[didx.at[j + 1]], rbs[q], sbs[q])
                ra = ras[p]
                rb = rbs[p]

                def row(i, carry2):
                    for qq in range(H // 16):
                        av = ra[i, pl.ds(qq * 16, 16)]
                        bv = rb[i, pl.ds(qq * 16, 16)]
                        rc[i, pl.ds(qq * 16, 16)] = av + bv
                    return carry2

                lax.fori_loop(0, ECH, row, 0)
                pltpu.sync_copy(
                    rc, out_hbm.at[pl.ds(e0 + (gi * EIG + j) * ECH, ECH)])
            return carry

        lax.fori_loop(0, ENG, grp, 0)

    return k(g, srcm, dstm)


def kernel(x, edge_index, edge_attr, We1, be1, We2, be2,
           Wl1, bl1, Wr1, Wl2, bl2, Wr2, Wo1, bo1, Wo2, bo2):
    src = edge_index[0].astype(jnp.int32)
    dst = edge_index[1].astype(jnp.int32)
    npadv = EPAD - E
    padv = N + (jnp.arange(npadv, dtype=jnp.int32) % 16)
    srcp = jnp.concatenate([src, padv])
    dstp = jnp.concatenate([dst, padv])
    srcm_s = srcp.reshape(EPAD // SCH, SCH)
    dstm_s = dstp.reshape(EPAD // SCH, SCH)
    srcm_e = srcp.reshape(EPAD // ECH, ECH)
    dstm_e = dstp.reshape(EPAD // ECH, ECH)

    xp = jnp.pad(x, ((0, NPAD - N), (0, 0)))
    ztab = jnp.zeros((NPAD, TW), f32)

    bl1r = bl1.reshape(1, H)
    bl2r = bl2.reshape(1, H)
    be1r = be1.reshape(1, H)
    be2r = be2.reshape(1, H)
    bo1r = bo1.reshape(1, H)
    bo2r = bo2.reshape(1, 1)

    y1t, r1 = _node1(xp, Wl1, Wr1, bl1r)
    acc1 = _seg_sum(y1t, srcm_s, dstm_s, ztab)
    y2t, r2 = _comb1(acc1, r1, Wl2, Wr2, bl2r)
    acc2 = _seg_sum(y2t, srcm_s, dstm_s, ztab)
    g = _comb2(acc2, r2, Wo1)
    efp = _edge_feats(g, srcm_e, dstm_e)
    out = _final(edge_attr, efp, We1, be1r, We2, Wo1, bo1r, be2r, Wo2, bo2r)
    return out[:, 0]


# speedup vs baseline: 5.2296x; 1.0695x over previous
import functools

import jax
import jax.numpy as jnp
from jax import lax
from jax.experimental import pallas as pl
from jax.experimental.pallas import tpu as pltpu
from jax.experimental.pallas import tpu_sc as plsc

N = 10000
E = 320000
D = 128
H = 64

NPAD = 10240
TW = 80
GW = 128
NTILES = 32
EPT = 10240
EPAD = EPT * NTILES
RPT = NPAD // 16

SCH = 128
SNC = EPT // SCH
SIG = 16
SNG = SNC // SIG

ECH = 128
ENC = EPT // ECH
EIG = 16
ENG = ENC // EIG

f32 = jnp.float32

_SC_PARAMS = pltpu.CompilerParams(use_tc_tiling_on_sc=False)


def _mktab(y, w):
    m = y.shape[0]
    return jnp.concatenate(
        [y, jnp.ones((m, 1), f32), jnp.zeros((m, w - H - 1), f32)], axis=1)


def _node1_body(x_ref, wl_ref, wr_ref, bl_ref, yt_ref, r_ref):
    x = x_ref[...]
    y = jnp.dot(x, wl_ref[...], preferred_element_type=f32)
    yt_ref[...] = _mktab(y, TW)
    r_ref[...] = jnp.dot(x, wr_ref[...], preferred_element_type=f32) + bl_ref[...]


def _node1(xp, Wl1, Wr1, bl1):
    G = NPAD // 512
    return pl.pallas_call(
        _node1_body,
        grid=(G,),
        in_specs=[
            pl.BlockSpec((512, D), lambda i: (i, 0)),
            pl.BlockSpec((D, H), lambda i: (0, 0)),
            pl.BlockSpec((D, H), lambda i: (0, 0)),
            pl.BlockSpec((1, H), lambda i: (0, 0)),
        ],
        out_specs=[
            pl.BlockSpec((512, TW), lambda i: (i, 0)),
            pl.BlockSpec((512, H), lambda i: (i, 0)),
        ],
        out_shape=[
            jax.ShapeDtypeStruct((NPAD, TW), f32),
            jax.ShapeDtypeStruct((NPAD, H), f32),
        ],
    )(xp, Wl1, Wr1, bl1)


def _comb_body(a0_ref, a1_ref, r_ref, wl_ref, wr_ref, bl_ref, yt_ref, r2_ref):
    a = a0_ref[...] + a1_ref[...]
    deg = jnp.maximum(a[:, H:H + 1], 1.0)
    h = jnp.maximum(a[:, :H] / deg + r_ref[...], 0.0)
    y2 = jnp.dot(h, wl_ref[...], preferred_element_type=f32)
    yt_ref[...] = _mktab(y2, TW)
    r2_ref[...] = jnp.dot(h, wr_ref[...], preferred_element_type=f32) + bl_ref[...]


def _comb1(acc, r1, Wl2, Wr2, bl2):
    G = NPAD // 512
    return pl.pallas_call(
        _comb_body,
        grid=(G,),
        in_specs=[
            pl.BlockSpec((512, TW), lambda i: (i, 0)),
            pl.BlockSpec((512, TW), lambda i: (i + NPAD // 512, 0)),
            pl.BlockSpec((512, H), lambda i: (i, 0)),
            pl.BlockSpec((H, H), lambda i: (0, 0)),
            pl.BlockSpec((H, H), lambda i: (0, 0)),
            pl.BlockSpec((1, H), lambda i: (0, 0)),
        ],
        out_specs=[
            pl.BlockSpec((512, TW), lambda i: (i, 0)),
            pl.BlockSpec((512, H), lambda i: (i, 0)),
        ],
        out_shape=[
            jax.ShapeDtypeStruct((NPAD, TW), f32),
            jax.ShapeDtypeStruct((NPAD, H), f32),
        ],
    )(acc, acc, r1, Wl2, Wr2, bl2)


def _comb2_body(a0_ref, a1_ref, r_ref, wo_ref, g_ref):
    a = a0_ref[...] + a1_ref[...]
    deg = jnp.maximum(a[:, H:H + 1], 1.0)
    h = jnp.maximum(a[:, :H] / deg + r_ref[...], 0.0)
    g = jnp.dot(h, wo_ref[...], preferred_element_type=f32)
    g_ref[...] = jnp.concatenate([g, jnp.zeros((g.shape[0], GW - H), f32)], axis=1)


def _comb2(acc, r2, Wo1):
    G = NPAD // 512
    return pl.pallas_call(
        _comb2_body,
        grid=(G,),
        in_specs=[
            pl.BlockSpec((512, TW), lambda i: (i, 0)),
            pl.BlockSpec((512, TW), lambda i: (i + NPAD // 512, 0)),
            pl.BlockSpec((512, H), lambda i: (i, 0)),
            pl.BlockSpec((H, H), lambda i: (0, 0)),
        ],
        out_specs=pl.BlockSpec((512, GW), lambda i: (i, 0)),
        out_shape=jax.ShapeDtypeStruct((NPAD, GW), f32),
    )(acc, acc, r2, Wo1)


def _mlp_body(ea_ref, we1_ref, be1_ref, we2_ref, wo1_ref,
              bo1_ref, be2_ref, z_ref):
    t = jnp.maximum(
        jnp.dot(ea_ref[...], we1_ref[...], preferred_element_type=f32)
        + be1_ref[...], 0.0)
    w2o = jnp.dot(we2_ref[...], wo1_ref[...], preferred_element_type=f32)
    cvec = jnp.dot(be2_ref[...], wo1_ref[...], preferred_element_type=f32) + bo1_ref[...]
    z_ref[...] = jnp.dot(t, w2o, preferred_element_type=f32) + cvec


def _edge_mlp(edge_attr, We1, be1, We2, Wo1, bo1, be2):
    BE = 2560
    G = E // BE
    return pl.pallas_call(
        _mlp_body,
        grid=(G,),
        in_specs=[
            pl.BlockSpec((BE, 16), lambda i: (i, 0)),
            pl.BlockSpec((16, H), lambda i: (0, 0)),
            pl.BlockSpec((1, H), lambda i: (0, 0)),
            pl.BlockSpec((H, H), lambda i: (0, 0)),
            pl.BlockSpec((H, H), lambda i: (0, 0)),
            pl.BlockSpec((1, H), lambda i: (0, 0)),
            pl.BlockSpec((1, H), lambda i: (0, 0)),
        ],
        out_specs=pl.BlockSpec((BE, H), lambda i: (i, 0)),
        out_shape=jax.ShapeDtypeStruct((E, H), f32),
    )(edge_attr, We1, be1, We2, Wo1, bo1, be2)


def _out_body(z_ref, efp_ref, wo2_ref, bo2_ref, o_ref):
    z = jnp.maximum(z_ref[...] + efp_ref[...], 0.0)
    o = jnp.dot(z, wo2_ref[...], preferred_element_type=f32) + bo2_ref[...]
    o_ref[...] = jax.nn.sigmoid(o)


def _out_stage(zpre, efp, Wo2, bo2):
    BE = 2560
    G = E // BE
    return pl.pallas_call(
        _out_body,
        grid=(G,),
        in_specs=[
            pl.BlockSpec((BE, H), lambda i: (i, 0)),
            pl.BlockSpec((BE, H), lambda i: (i, 0)),
            pl.BlockSpec((H, 1), lambda i: (0, 0)),
            pl.BlockSpec((1, 1), lambda i: (0, 0)),
        ],
        out_specs=pl.BlockSpec((BE, 1), lambda i: (i, 0)),
        out_shape=jax.ShapeDtypeStruct((E, 1), f32),
    )(zpre, efp, Wo2, bo2)


_MESH = dict(core_axis_name="c", subcore_axis_name="s")


def _seg_sum(table, srcm, dstm, ztab):
    mesh = plsc.VectorSubcoreMesh(**_MESH)

    @functools.partial(
        pl.kernel,
        out_type=jax.ShapeDtypeStruct((2 * NPAD, TW), f32),
        mesh=mesh,
        compiler_params=_SC_PARAMS,
        scratch_types=[
            pltpu.VMEM_SHARED((NPAD, TW), f32),
            pltpu.VMEM((SIG, SCH), jnp.int32),
            pltpu.VMEM((SIG, SCH), jnp.int32),
            pltpu.VMEM((SCH, TW), f32),
            pltpu.VMEM((SCH, TW), f32),
            pltpu.SemaphoreType.DMA,
            pltpu.SemaphoreType.DMA,
        ],
    )
    def k(table_hbm, src_hbm, dst_hbm, z_hbm, out_hbm,
          acc_sh, sidx, didx, rows0, rows1, sem0, sem1):
        c = lax.axis_index("c")
        s = lax.axis_index("s")
        r0 = s * RPT
        pltpu.sync_copy(z_hbm.at[pl.ds(r0, RPT)], acc_sh.at[pl.ds(r0, RPT)])
        wid = c * 16 + s
        cb = wid * SNC
        plsc.subcore_barrier()

        rbufs = (rows0, rows1)
        sems = (sem0, sem1)

        def grp(gi, carry):
            pltpu.sync_copy(src_hbm.at[pl.ds(cb + gi * SIG, SIG)], sidx)
            pltpu.sync_copy(dst_hbm.at[pl.ds(cb + gi * SIG, SIG)], didx)
            copies = [None, None]
            copies[0] = pltpu.async_copy(
                table_hbm.at[sidx.at[0]], rbufs[0], sems[0])
            for j in range(SIG):
                p = j % 2
                copies[p].wait()
                if j + 1 < SIG:
                    q = (j + 1) % 2
                    copies[q] = pltpu.async_copy(
                        table_hbm.at[sidx.at[j + 1]], rbufs[q], sems[q])
                pltpu.sync_copy(rbufs[p], acc_sh.at[didx.at[j]], add=True)
            return carry

        lax.fori_loop(0, SNG, grp, 0)
        plsc.subcore_barrier()
        pltpu.sync_copy(acc_sh.at[pl.ds(r0, RPT)],
                        out_hbm.at[pl.ds(c * NPAD + r0, RPT)])

    return k(table, srcm, dstm, ztab)


def _edge_feats(g, srcm, dstm):
    mesh = plsc.VectorSubcoreMesh(**_MESH)

    @functools.partial(
        pl.kernel,
        out_type=jax.ShapeDtypeStruct((EPAD, H), f32),
        mesh=mesh,
        scratch_types=[
            pltpu.VMEM((EIG, ECH), jnp.int32),
            pltpu.VMEM((EIG, ECH), jnp.int32),
            pltpu.VMEM((ECH, GW), f32),
            pltpu.VMEM((ECH, GW), f32),
            pltpu.VMEM((ECH, GW), f32),
            pltpu.VMEM((ECH, GW), f32),
            pltpu.VMEM((ECH, H), f32),
            pltpu.SemaphoreType.DMA,
            pltpu.SemaphoreType.DMA,
            pltpu.SemaphoreType.DMA,
            pltpu.SemaphoreType.DMA,
        ],
    )
    def k(g_hbm, src_hbm, dst_hbm, out_hbm,
          sidx, didx, ra0, rb0, ra1, rb1, rc, sa0, sb0, sa1, sb1):
        c = lax.axis_index("c")
        s = lax.axis_index("s")
        wid = c * 16 + s
        cb = wid * ENC
        e0 = wid * EPT

        ras = (ra0, ra1)
        rbs = (rb0, rb1)
        sas = (sa0, sa1)
        sbs = (sb0, sb1)

        def grp(gi, carry):
            pltpu.sync_copy(src_hbm.at[pl.ds(cb + gi * EIG, EIG)], sidx)
            pltpu.sync_copy(dst_hbm.at[pl.ds(cb + gi * EIG, EIG)], didx)
            ca = [None, None]
            cbq = [None, None]
            ca[0] = pltpu.async_copy(g_hbm.at[sidx.at[0]], ras[0], sas[0])
            cbq[0] = pltpu.async_copy(g_hbm.at[didx.at[0]], rbs[0], sbs[0])
            for j in range(EIG):
                p = j % 2
                ca[p].wait()
                cbq[p].wait()
                if j + 1 < EIG:
                    q = (j + 1) % 2
                    ca[q] = pltpu.async_copy(
                        g_hbm.at[sidx.at[j + 1]], ras[q], sas[q])
                    cbq[q] = pltpu.async_copy(
                        g_hbm.at[didx.at[j + 1]], rbs[q], sbs[q])
                ra = ras[p]
                rb = rbs[p]

                def row(i, carry2):
                    for qq in range(H // 16):
                        av = ra[i, pl.ds(qq * 16, 16)]
                        bv = rb[i, pl.ds(qq * 16, 16)]
                        rc[i, pl.ds(qq * 16, 16)] = av + bv
                    return carry2

                lax.fori_loop(0, ECH, row, 0)
                pltpu.sync_copy(
                    rc, out_hbm.at[pl.ds(e0 + (gi * EIG + j) * ECH, ECH)])
            return carry

        lax.fori_loop(0, ENG, grp, 0)

    return k(g, srcm, dstm)


def kernel(x, edge_index, edge_attr, We1, be1, We2, be2,
           Wl1, bl1, Wr1, Wl2, bl2, Wr2, Wo1, bo1, Wo2, bo2):
    src = edge_index[0].astype(jnp.int32)
    dst = edge_index[1].astype(jnp.int32)
    npadv = EPAD - E
    padv = N + (jnp.arange(npadv, dtype=jnp.int32) % 16)
    srcp = jnp.concatenate([src, padv])
    dstp = jnp.concatenate([dst, padv])
    srcm_s = srcp.reshape(EPAD // SCH, SCH)
    dstm_s = dstp.reshape(EPAD // SCH, SCH)
    srcm_e = srcp.reshape(EPAD // ECH, ECH)
    dstm_e = dstp.reshape(EPAD // ECH, ECH)

    xp = jnp.pad(x, ((0, NPAD - N), (0, 0)))
    ztab = jnp.zeros((NPAD, TW), f32)

    bl1r = bl1.reshape(1, H)
    bl2r = bl2.reshape(1, H)
    be1r = be1.reshape(1, H)
    be2r = be2.reshape(1, H)
    bo1r = bo1.reshape(1, H)
    bo2r = bo2.reshape(1, 1)

    zpre = _edge_mlp(edge_attr, We1, be1r, We2, Wo1, bo1r, be2r)
    y1t, r1 = _node1(xp, Wl1, Wr1, bl1r)
    acc1 = _seg_sum(y1t, srcm_s, dstm_s, ztab)
    y2t, r2 = _comb1(acc1, r1, Wl2, Wr2, bl2r)
    acc2 = _seg_sum(y2t, srcm_s, dstm_s, ztab)
    g = _comb2(acc2, r2, Wo1)
    efp = _edge_feats(g, srcm_e, dstm_e)
    out = _out_stage(zpre, efp, Wo2, bo2r)
    return out[:, 0]


# speedup vs baseline: 5.9924x; 1.1459x over previous
import functools

import jax
import jax.numpy as jnp
from jax import lax
from jax.experimental import pallas as pl
from jax.experimental.pallas import tpu as pltpu
from jax.experimental.pallas import tpu_sc as plsc

N = 10000
E = 320000
D = 128
H = 64

NPAD = 10240
TW = 80
GW = 128
NTILES = 32
EPT = 10240
EPAD = EPT * NTILES
RPT = NPAD // 16

SCH = 128
SNC = EPT // SCH
SIG = 16
SNG = SNC // SIG

ECH = 128
ENC = EPT // ECH
EIG = 16
ENG = ENC // EIG

f32 = jnp.float32

_SC_PARAMS = pltpu.CompilerParams(use_tc_tiling_on_sc=False)


def _mktab(y, w):
    m = y.shape[0]
    return jnp.concatenate(
        [y, jnp.ones((m, 1), f32), jnp.zeros((m, w - H - 1), f32)], axis=1)


def _node1_body(x_ref, wl_ref, wr_ref, bl_ref, yt_ref, r_ref):
    x = x_ref[...]
    y = jnp.dot(x, wl_ref[...], preferred_element_type=f32)
    yt_ref[...] = _mktab(y, TW)
    r_ref[...] = jnp.dot(x, wr_ref[...], preferred_element_type=f32) + bl_ref[...]


def _node1(xp, Wl1, Wr1, bl1):
    G = NPAD // 512
    return pl.pallas_call(
        _node1_body,
        grid=(G,),
        in_specs=[
            pl.BlockSpec((512, D), lambda i: (i, 0)),
            pl.BlockSpec((D, H), lambda i: (0, 0)),
            pl.BlockSpec((D, H), lambda i: (0, 0)),
            pl.BlockSpec((1, H), lambda i: (0, 0)),
        ],
        out_specs=[
            pl.BlockSpec((512, TW), lambda i: (i, 0)),
            pl.BlockSpec((512, H), lambda i: (i, 0)),
        ],
        out_shape=[
            jax.ShapeDtypeStruct((NPAD, TW), f32),
            jax.ShapeDtypeStruct((NPAD, H), f32),
        ],
    )(xp, Wl1, Wr1, bl1)


def _comb_body(a0_ref, a1_ref, r_ref, wl_ref, wr_ref, bl_ref, yt_ref, r2_ref):
    a = a0_ref[...] + a1_ref[...]
    deg = jnp.maximum(a[:, H:H + 1], 1.0)
    h = jnp.maximum(a[:, :H] / deg + r_ref[...], 0.0)
    y2 = jnp.dot(h, wl_ref[...], preferred_element_type=f32)
    yt_ref[...] = _mktab(y2, TW)
    r2_ref[...] = jnp.dot(h, wr_ref[...], preferred_element_type=f32) + bl_ref[...]


def _comb1(acc, r1, Wl2, Wr2, bl2):
    G = NPAD // 512
    return pl.pallas_call(
        _comb_body,
        grid=(G,),
        in_specs=[
            pl.BlockSpec((512, TW), lambda i: (i, 0)),
            pl.BlockSpec((512, TW), lambda i: (i + NPAD // 512, 0)),
            pl.BlockSpec((512, H), lambda i: (i, 0)),
            pl.BlockSpec((H, H), lambda i: (0, 0)),
            pl.BlockSpec((H, H), lambda i: (0, 0)),
            pl.BlockSpec((1, H), lambda i: (0, 0)),
        ],
        out_specs=[
            pl.BlockSpec((512, TW), lambda i: (i, 0)),
            pl.BlockSpec((512, H), lambda i: (i, 0)),
        ],
        out_shape=[
            jax.ShapeDtypeStruct((NPAD, TW), f32),
            jax.ShapeDtypeStruct((NPAD, H), f32),
        ],
    )(acc, acc, r1, Wl2, Wr2, bl2)


def _comb2_body(a0_ref, a1_ref, r_ref, wo_ref, g_ref):
    a = a0_ref[...] + a1_ref[...]
    deg = jnp.maximum(a[:, H:H + 1], 1.0)
    h = jnp.maximum(a[:, :H] / deg + r_ref[...], 0.0)
    g = jnp.dot(h, wo_ref[...], preferred_element_type=f32)
    g_ref[...] = jnp.concatenate([g, jnp.zeros((g.shape[0], GW - H), f32)], axis=1)


def _comb2(acc, r2, Wo1):
    G = NPAD // 512
    return pl.pallas_call(
        _comb2_body,
        grid=(G,),
        in_specs=[
            pl.BlockSpec((512, TW), lambda i: (i, 0)),
            pl.BlockSpec((512, TW), lambda i: (i + NPAD // 512, 0)),
            pl.BlockSpec((512, H), lambda i: (i, 0)),
            pl.BlockSpec((H, H), lambda i: (0, 0)),
        ],
        out_specs=pl.BlockSpec((512, GW), lambda i: (i, 0)),
        out_shape=jax.ShapeDtypeStruct((NPAD, GW), f32),
    )(acc, acc, r2, Wo1)


def _mlp_body(ea_ref, we1_ref, be1_ref, we2_ref, wo1_ref,
              bo1_ref, be2_ref, z_ref):
    t = jnp.maximum(
        lax.dot_general(ea_ref[...], we1_ref[...],
                        (((0,), (0,)), ((), ())),
                        preferred_element_type=f32)
        + be1_ref[...], 0.0)
    w2o = jnp.dot(we2_ref[...], wo1_ref[...], preferred_element_type=f32)
    cvec = jnp.dot(be2_ref[...], wo1_ref[...], preferred_element_type=f32) + bo1_ref[...]
    z_ref[...] = jnp.dot(t, w2o, preferred_element_type=f32) + cvec


def _edge_mlp(eaT, We1, be1, We2, Wo1, bo1, be2):
    BE = 2560
    G = E // BE
    return pl.pallas_call(
        _mlp_body,
        grid=(G,),
        in_specs=[
            pl.BlockSpec((16, BE), lambda i: (0, i)),
            pl.BlockSpec((16, H), lambda i: (0, 0)),
            pl.BlockSpec((1, H), lambda i: (0, 0)),
            pl.BlockSpec((H, H), lambda i: (0, 0)),
            pl.BlockSpec((H, H), lambda i: (0, 0)),
            pl.BlockSpec((1, H), lambda i: (0, 0)),
            pl.BlockSpec((1, H), lambda i: (0, 0)),
        ],
        out_specs=pl.BlockSpec((BE, H), lambda i: (i, 0)),
        out_shape=jax.ShapeDtypeStruct((E, H), f32),
    )(eaT, We1, be1, We2, Wo1, bo1, be2)


def _out_body(z_ref, efp_ref, wo2_ref, bo2_ref, o_ref):
    z = jnp.maximum(z_ref[...] + efp_ref[...], 0.0)
    o2 = jnp.dot(z, wo2_ref[...], preferred_element_type=f32) + bo2_ref[0, 0]
    o = jnp.reshape(jax.nn.sigmoid(o2), (z.shape[0],))
    i = pl.program_id(0)
    o_ref[pl.ds(i * z.shape[0], z.shape[0])] = o


def _out_stage(zpre, efp, Wo2, bo2):
    BE = 2560
    G = E // BE
    return pl.pallas_call(
        _out_body,
        grid=(G,),
        in_specs=[
            pl.BlockSpec((BE, H), lambda i: (i, 0)),
            pl.BlockSpec((BE, H), lambda i: (i, 0)),
            pl.BlockSpec((H, 1), lambda i: (0, 0)),
            pl.BlockSpec((1, 1), lambda i: (0, 0)),
        ],
        out_specs=pl.BlockSpec((E,), lambda i: (0,)),
        out_shape=jax.ShapeDtypeStruct((E,), f32),
    )(zpre, efp, Wo2, bo2)


_MESH = dict(core_axis_name="c", subcore_axis_name="s")


def _seg_sum(table, srcm, dstm, ztab):
    mesh = plsc.VectorSubcoreMesh(**_MESH)

    @functools.partial(
        pl.kernel,
        out_type=jax.ShapeDtypeStruct((2 * NPAD, TW), f32),
        mesh=mesh,
        compiler_params=_SC_PARAMS,
        scratch_types=[
            pltpu.VMEM_SHARED((NPAD, TW), f32),
            pltpu.VMEM((SIG, SCH), jnp.int32),
            pltpu.VMEM((SIG, SCH), jnp.int32),
            pltpu.VMEM((SCH, TW), f32),
            pltpu.VMEM((SCH, TW), f32),
            pltpu.SemaphoreType.DMA,
            pltpu.SemaphoreType.DMA,
        ],
    )
    def k(table_hbm, src_hbm, dst_hbm, z_hbm, out_hbm,
          acc_sh, sidx, didx, rows0, rows1, sem0, sem1):
        c = lax.axis_index("c")
        s = lax.axis_index("s")
        r0 = s * RPT
        pltpu.sync_copy(z_hbm.at[pl.ds(r0, RPT)], acc_sh.at[pl.ds(r0, RPT)])
        wid = c * 16 + s
        cb = wid * SNC
        plsc.subcore_barrier()

        rbufs = (rows0, rows1)
        sems = (sem0, sem1)

        def grp(gi, carry):
            pltpu.sync_copy(src_hbm.at[pl.ds(cb + gi * SIG, SIG)], sidx)
            pltpu.sync_copy(dst_hbm.at[pl.ds(cb + gi * SIG, SIG)], didx)
            copies = [None, None]
            copies[0] = pltpu.async_copy(
                table_hbm.at[sidx.at[0]], rbufs[0], sems[0])
            for j in range(SIG):
                p = j % 2
                copies[p].wait()
                if j + 1 < SIG:
                    q = (j + 1) % 2
                    copies[q] = pltpu.async_copy(
                        table_hbm.at[sidx.at[j + 1]], rbufs[q], sems[q])
                pltpu.sync_copy(rbufs[p], acc_sh.at[didx.at[j]], add=True)
            return carry

        lax.fori_loop(0, SNG, grp, 0)
        plsc.subcore_barrier()
        pltpu.sync_copy(acc_sh.at[pl.ds(r0, RPT)],
                        out_hbm.at[pl.ds(c * NPAD + r0, RPT)])

    return k(table, srcm, dstm, ztab)


def _edge_feats(g, srcm, dstm):
    mesh = plsc.VectorSubcoreMesh(**_MESH)

    @functools.partial(
        pl.kernel,
        out_type=jax.ShapeDtypeStruct((EPAD, H), f32),
        mesh=mesh,
        scratch_types=[
            pltpu.VMEM((EIG, ECH), jnp.int32),
            pltpu.VMEM((EIG, ECH), jnp.int32),
            pltpu.VMEM((ECH, GW), f32),
            pltpu.VMEM((ECH, GW), f32),
            pltpu.VMEM((ECH, GW), f32),
            pltpu.VMEM((ECH, GW), f32),
            pltpu.VMEM((ECH, H), f32),
            pltpu.SemaphoreType.DMA,
            pltpu.SemaphoreType.DMA,
            pltpu.SemaphoreType.DMA,
            pltpu.SemaphoreType.DMA,
        ],
    )
    def k(g_hbm, src_hbm, dst_hbm, out_hbm,
          sidx, didx, ra0, rb0, ra1, rb1, rc, sa0, sb0, sa1, sb1):
        c = lax.axis_index("c")
        s = lax.axis_index("s")
        wid = c * 16 + s
        cb = wid * ENC
        e0 = wid * EPT

        ras = (ra0, ra1)
        rbs = (rb0, rb1)
        sas = (sa0, sa1)
        sbs = (sb0, sb1)

        def grp(gi, carry):
            pltpu.sync_copy(src_hbm.at[pl.ds(cb + gi * EIG, EIG)], sidx)
            pltpu.sync_copy(dst_hbm.at[pl.ds(cb + gi * EIG, EIG)], didx)
            ca = [None, None]
            cbq = [None, None]
            ca[0] = pltpu.async_copy(g_hbm.at[sidx.at[0]], ras[0], sas[0])
            cbq[0] = pltpu.async_copy(g_hbm.at[didx.at[0]], rbs[0], sbs[0])
            for j in range(EIG):
                p = j % 2
                ca[p].wait()
                cbq[p].wait()
                if j + 1 < EIG:
                    q = (j + 1) % 2
                    ca[q] = pltpu.async_copy(
                        g_hbm.at[sidx.at[j + 1]], ras[q], sas[q])
                    cbq[q] = pltpu.async_copy(
                        g_hbm.at[didx.at[j + 1]], rbs[q], sbs[q])
                ra = ras[p]
                rb = rbs[p]

                def row(i, carry2):
                    for qq in range(H // 16):
                        av = ra[i, pl.ds(qq * 16, 16)]
                        bv = rb[i, pl.ds(qq * 16, 16)]
                        rc[i, pl.ds(qq * 16, 16)] = av + bv
                    return carry2

                lax.fori_loop(0, ECH, row, 0)
                pltpu.sync_copy(
                    rc, out_hbm.at[pl.ds(e0 + (gi * EIG + j) * ECH, ECH)])
            return carry

        lax.fori_loop(0, ENG, grp, 0)

    return k(g, srcm, dstm)


def kernel(x, edge_index, edge_attr, We1, be1, We2, be2,
           Wl1, bl1, Wr1, Wl2, bl2, Wr2, Wo1, bo1, Wo2, bo2):
    src = edge_index[0].astype(jnp.int32)
    dst = edge_index[1].astype(jnp.int32)
    npadv = EPAD - E
    padv = N + (jnp.arange(npadv, dtype=jnp.int32) % 16)
    srcp = jnp.concatenate([src, padv])
    dstp = jnp.concatenate([dst, padv])
    srcm_s = srcp.reshape(EPAD // SCH, SCH)
    dstm_s = dstp.reshape(EPAD // SCH, SCH)
    srcm_e = srcp.reshape(EPAD // ECH, ECH)
    dstm_e = dstp.reshape(EPAD // ECH, ECH)

    xp = jnp.pad(x, ((0, NPAD - N), (0, 0)))
    ztab = jnp.zeros((NPAD, TW), f32)

    bl1r = bl1.reshape(1, H)
    bl2r = bl2.reshape(1, H)
    be1r = be1.reshape(1, H)
    be2r = be2.reshape(1, H)
    bo1r = bo1.reshape(1, H)
    bo2r = bo2.reshape(1, 1)

    zpre = _edge_mlp(edge_attr.T, We1, be1r, We2, Wo1, bo1r, be2r)
    y1t, r1 = _node1(xp, Wl1, Wr1, bl1r)
    acc1 = _seg_sum(y1t, srcm_s, dstm_s, ztab)
    y2t, r2 = _comb1(acc1, r1, Wl2, Wr2, bl2r)
    acc2 = _seg_sum(y2t, srcm_s, dstm_s, ztab)
    g = _comb2(acc2, r2, Wo1)
    efp = _edge_feats(g, srcm_e, dstm_e)
    return _out_stage(zpre, efp, Wo2, bo2r)
